# baseline (speedup 1.0000x reference)
import functools

import jax
import jax.numpy as jnp
from jax.experimental import pallas as pl
from jax.experimental.pallas import tpu as pltpu

CODEBOOK_SIZE = 1024
D_MODEL = 256
COMMITMENT_COST = 0.25


def _vq_body(x1_ref, x3_ref, cb_ref, q_ref, idx_ref, loss_ref, perp_ref,
             dist_scr, oh_scr, counts_ref, acc_ref, *, n_tokens):
    i = pl.program_id(0)
    cur = jax.lax.rem(i, 2)
    alt = 1 - cur
    cb = cb_ref[...]

    oh3 = oh_scr[cur]
    x3 = x3_ref[...]
    q = jax.lax.dot_general(
        oh3, cb.astype(jnp.bfloat16), (((1,), (0,)), ((), ())),
        preferred_element_type=jnp.float32)
    q_ref[...] = q
    part_loss = jnp.sum((q - x3) ** 2)
    ones_row = jnp.ones((8, x3.shape[0]), jnp.bfloat16)
    part_counts = jax.lax.dot_general(
        ones_row, oh3, (((1,), (0,)), ((), ())),
        preferred_element_type=jnp.float32)[:1]

    valid = i >= 2
    acc_new = jnp.where(valid, acc_ref[0] + part_loss, 0.0)
    acc_ref[0] = acc_new
    counts_new = jnp.where(valid, counts_ref[...] + part_counts,
                           jnp.zeros_like(part_counts))
    counts_ref[...] = counts_new

    loss_ref[...] = jnp.reshape(
        acc_new * (COMMITMENT_COST / (n_tokens * D_MODEL)), (1, 1))
    p = counts_new / n_tokens
    perp_ref[...] = jnp.reshape(
        jnp.exp(-jnp.sum(p * jnp.log(p + 1e-10))), (1, 1))

    d = dist_scr[alt]
    iota = jax.lax.broadcasted_iota(jnp.int32, d.shape, 1)
    minv = jnp.min(d, axis=1, keepdims=True)
    idxv = jnp.min(jnp.where(d == minv, iota, CODEBOOK_SIZE), axis=1)
    idx_ref[0, 0, :] = idxv
    oh_scr[alt] = (iota == idxv[:, None]).astype(jnp.float32).astype(jnp.bfloat16)

    x1 = x1_ref[...]
    xsq = jnp.sum(x1 * x1, axis=1, keepdims=True)
    csq = jnp.sum(cb * cb, axis=1)
    dotsm2 = jax.lax.dot_general(
        x1, cb * -2.0, (((1,), (1,)), ((), ())),
        preferred_element_type=jnp.float32)
    dist_scr[cur] = (xsq + csq[None, :]) + dotsm2


def kernel(inputs, codebook, temp):
    del temp
    input_shape = inputs.shape
    x = inputs.reshape(-1, D_MODEL)
    n_tokens = x.shape[0]
    tile = 1024
    n_tiles = n_tokens // tile
    n_steps = n_tiles + 2
    last = n_tiles - 1

    q, idx, loss, perp = pl.pallas_call(
        functools.partial(_vq_body, n_tokens=n_tokens),
        grid=(n_steps,),
        in_specs=[
            pl.BlockSpec((tile, D_MODEL),
                         lambda i: (jnp.minimum(i, last), 0)),
            pl.BlockSpec((tile, D_MODEL),
                         lambda i: (jnp.clip(i - 2, 0, last), 0)),
            pl.BlockSpec((CODEBOOK_SIZE, D_MODEL), lambda i: (0, 0)),
        ],
        out_specs=[
            pl.BlockSpec((tile, D_MODEL),
                         lambda i: (jnp.clip(i - 2, 0, last), 0)),
            pl.BlockSpec((1, 1, tile),
                         lambda i: (jnp.clip(i - 1, 0, last), 0, 0)),
            pl.BlockSpec((1, 1), lambda i: (0, 0)),
            pl.BlockSpec((1, 1), lambda i: (0, 0)),
        ],
        out_shape=[
            jax.ShapeDtypeStruct((n_tokens, D_MODEL), jnp.float32),
            jax.ShapeDtypeStruct((n_tiles, 1, tile), jnp.int32),
            jax.ShapeDtypeStruct((1, 1), jnp.float32),
            jax.ShapeDtypeStruct((1, 1), jnp.float32),
        ],
        scratch_shapes=[
            pltpu.VMEM((2, tile, CODEBOOK_SIZE), jnp.float32),
            pltpu.VMEM((2, tile, CODEBOOK_SIZE), jnp.bfloat16),
            pltpu.VMEM((1, CODEBOOK_SIZE), jnp.float32),
            pltpu.SMEM((1,), jnp.float32),
        ],
    )(x, x, codebook)

    return (q.reshape(input_shape),
            idx.reshape(input_shape[:-1]),
            loss.reshape(()),
            perp.reshape(()))
